# baseline (speedup 1.0000x reference)
import jax
import jax.numpy as jnp
from jax.experimental import pallas as pl
from jax.experimental.pallas import tpu as pltpu

B = 32
T = 64
F = 128
H = 256
M = 5
OUT = 128
E_T = 512
E_S = 128
XIN = H + 2 * F
G = 8

_F32 = jnp.float32
_BF16 = jnp.bfloat16


def _dot(a, b):
    return jax.lax.dot_general(a, b, (((1,), (0,)), ((), ())),
                               preferred_element_type=_F32)


def _dot_t(a, b):
    return jax.lax.dot_general(a, b, (((1,), (1,)), ((), ())),
                               preferred_element_type=_F32)


def _dot_lt(a, b):
    return jax.lax.dot_general(a, b, (((0,), (0,)), ((), ())),
                               preferred_element_type=_F32)


def _b16(x):
    return x.astype(_BF16)


def _graph_kernel(x_ref, tei_ref, tew_ref, ei_ref, ew_ref,
                  wt_ref, bt_ref, ws_ref, bs_ref,
                  xin_ref, afs_ref, abs_ref):
    b = pl.program_id(0)

    @pl.when(b == 0)
    def _():
        src = ei_ref[0:1, :].astype(jnp.int32)
        dst = ei_ref[1:2, :].astype(jnp.int32)
        w = ew_ref[...]
        iota = jax.lax.broadcasted_iota(jnp.int32, (F, E_S), 0)
        gs = (iota == src).astype(_F32)
        gd = (iota == dst).astype(_F32)
        deg_out = jnp.sum(gs * w, axis=1, keepdims=True)
        deg_in = jnp.sum(gd * w, axis=1, keepdims=True)
        dso = jnp.where(deg_out > 0, deg_out, 1.0)
        dsi = jnp.where(deg_in > 0, deg_in, 1.0)
        w_fwd = w / jnp.sum(gs * dso, axis=0, keepdims=True)
        w_bwd = w / jnp.sum(gd * dsi, axis=0, keepdims=True)
        afs_ref[...] = _b16(_dot_t(_b16(gs), _b16(gd * w_fwd)))
        abs_ref[...] = _b16(_dot_t(_b16(gd), _b16(gs * w_bwd)))

    afs = afs_ref[...]
    abs_ = abs_ref[...]

    for j in range(G):
        src = tei_ref[j, 0:1, :]
        dst = tei_ref[j, 1:2, :]
        w = tew_ref[j]
        iota = jax.lax.broadcasted_iota(jnp.int32, (T, E_T), 0)
        gs = (iota == src).astype(_F32)
        gd = (iota == dst).astype(_F32)
        deg_out = jnp.sum(gs * w, axis=1, keepdims=True)
        deg_in = jnp.sum(gd * w, axis=1, keepdims=True)
        dso = jnp.where(deg_out > 0, deg_out, 1.0)
        dsi = jnp.where(deg_in > 0, deg_in, 1.0)
        w_fwd = w / jnp.sum(gs * dso, axis=0, keepdims=True)
        w_bwd = w / jnp.sum(gd * dsi, axis=0, keepdims=True)
        a_f = _b16(_dot_t(_b16(gd), _b16(gs * w_fwd)))
        a_b = _b16(_dot_t(_b16(gs), _b16(gd * w_bwd)))

        xb = _b16(x_ref[j])
        zf1 = _b16(_dot(a_f, xb))
        zf2 = _b16(_dot(a_f, zf1))
        zb1 = _b16(_dot(a_b, xb))
        zb2 = _b16(_dot(a_b, zb1))
        dt = (_dot(zf1, wt_ref[0:F]) + _dot(zf2, wt_ref[F:2 * F])
              + _dot(zb1, wt_ref[2 * F:3 * F]) + _dot(zb2, wt_ref[3 * F:4 * F])
              + bt_ref[...])

        y1 = _b16(_dot(xb, afs))
        y2 = _b16(_dot(y1, afs))
        y3 = _b16(_dot(xb, abs_))
        y4 = _b16(_dot(y3, abs_))
        ds = (_dot(ws_ref[0], y1) + _dot(ws_ref[1], y2)
              + _dot(ws_ref[2], y3) + _dot(ws_ref[3], y4)
              + bs_ref[...])

        xin_ref[j] = jnp.concatenate([_b16(dt), _b16(ds), xb], axis=1)


def _proj_lstm_head_kernel(xin_ref, wih_ref, bg_ref, whh_ref,
                           muw_ref, mub_ref, sgw_ref, sgb_ref,
                           piw_ref, pib_ref,
                           mu_ref, sg_ref, pi_ref, p_ref, hs_ref):
    xin = xin_ref[...].reshape(T * B, XIN)
    p_ref[...] = (_dot(xin, wih_ref[...]) + bg_ref[...]).reshape(T, B, 4 * H)

    def _sig(x):
        return 0.5 * jnp.tanh(0.5 * x) + 0.5

    def step(t, carry):
        h, c = carry
        pt = p_ref[pl.ds(t, 1)].reshape(B, 4 * H)
        gates = pt + _dot(h, whh_ref[...])
        i = _sig(gates[:, 0:H])
        f = _sig(gates[:, H:2 * H])
        g = jnp.tanh(gates[:, 2 * H:3 * H])
        o = _sig(gates[:, 3 * H:4 * H])
        c2 = f * c + i * g
        h2 = o * jnp.tanh(c2)
        hs_ref[pl.ds(t, 1)] = h2.reshape(1, B, H)
        return (_b16(h2), c2)

    zeros16 = jnp.zeros((B, H), _BF16)
    zeros = jnp.zeros((B, H), _F32)
    jax.lax.fori_loop(0, T, step, (zeros16, zeros))

    hs = _b16(jnp.swapaxes(hs_ref[...], 0, 1).reshape(B * T, H))
    mu_ref[...] = _dot(hs, muw_ref[...]) + mub_ref[...]
    sg_ref[...] = jnp.exp(_dot(hs, sgw_ref[...]) + sgb_ref[...])
    logits = _dot(hs, piw_ref[...]) + pib_ref[...]
    mx = jnp.max(logits, axis=-1, keepdims=True)
    e = jnp.exp(logits - mx)
    pi_ref[...] = e / jnp.sum(e, axis=-1, keepdims=True)


def kernel(x, temporal_edge_i, temporal_edge_w, edge_index, edge_weight,
           Wt, bt, Ws, bs, W_ih, W_hh, b_ih, b_hh,
           mu_w, mu_b, sigma_w, sigma_b, pi_w, pi_b, interpret=False):
    bg = (b_ih + b_hh)[None, :]

    xin = pl.pallas_call(
        _graph_kernel,
        grid=(B // G,),
        in_specs=[
            pl.BlockSpec((G, T, F), lambda b: (b, 0, 0)),
            pl.BlockSpec((G, 2, E_T), lambda b: (b, 0, 0)),
            pl.BlockSpec((G, 1, E_T), lambda b: (b, 0, 0)),
            pl.BlockSpec((2, E_S), lambda b: (0, 0)),
            pl.BlockSpec((1, E_S), lambda b: (0, 0)),
            pl.BlockSpec((4 * F, H), lambda b: (0, 0)),
            pl.BlockSpec((1, H), lambda b: (0, 0)),
            pl.BlockSpec((4, T, T), lambda b: (0, 0, 0)),
            pl.BlockSpec((T, 1), lambda b: (0, 0)),
        ],
        out_specs=pl.BlockSpec((G, T, XIN), lambda b: (b, 0, 0)),
        out_shape=jax.ShapeDtypeStruct((B, T, XIN), _BF16),
        scratch_shapes=[pltpu.VMEM((F, F), _BF16), pltpu.VMEM((F, F), _BF16)],
        interpret=interpret,
    )(x, temporal_edge_i, temporal_edge_w[:, None, :], edge_index,
      edge_weight[None, :], _b16(Wt), bt[None, :],
      _b16(jnp.swapaxes(Ws.reshape(4, T, T), 1, 2)), bs[:, None])

    mu_f, sg_f, pi_f = pl.pallas_call(
        _proj_lstm_head_kernel,
        out_shape=[
            jax.ShapeDtypeStruct((B * T, M * OUT), _F32),
            jax.ShapeDtypeStruct((B * T, M * OUT), _F32),
            jax.ShapeDtypeStruct((B * T, M), _F32),
        ],
        scratch_shapes=[pltpu.VMEM((T, B, 4 * H), _F32),
                        pltpu.VMEM((T, B, H), _F32)],
        interpret=interpret,
    )(jnp.swapaxes(xin, 0, 1), _b16(W_ih.T), bg, _b16(W_hh.T),
      _b16(mu_w.T), mu_b[None, :], _b16(sigma_w.T), sigma_b[None, :],
      _b16(pi_w.T), pi_b[None, :])

    mu = mu_f.reshape(B, T, M, OUT)
    sigma = sg_f.reshape(B, T, M, OUT)
    pi = pi_f.reshape(B, T, M)
    return mu, sigma, pi
